# baseline (speedup 1.0000x reference)
import functools

import jax
import jax.numpy as jnp
from jax import lax
from jax.experimental import pallas as pl
from jax.experimental.pallas import tpu as pltpu
from jax.experimental.pallas import tpu_sc as plsc

B = 16384
E = 64
T = 128
V = 128
H = 128
F = 2 * E + T + V

NC = 2
NS = 16
NW = NC * NS
BPW = B // NW
IDX_ROWS = BPW // 128


def _sc_gather_body(uidx_hbm, ridx_hbm, utab_hbm, rtab_hbm,
                    uout_hbm, rout_hbm,
                    uidx_v, ridx_v, urows_v, rrows_v, sem_u, sem_r):
    wid = lax.axis_index("s") * NC + lax.axis_index("c")
    base = wid * BPW
    row0 = wid * IDX_ROWS
    pltpu.sync_copy(uidx_hbm.at[pl.ds(row0, IDX_ROWS)], uidx_v)
    pltpu.sync_copy(ridx_hbm.at[pl.ds(row0, IDX_ROWS)], ridx_v)
    copies = []
    for j in range(IDX_ROWS):
        copies.append(pltpu.async_copy(
            utab_hbm.at[uidx_v.at[j]], urows_v.at[pl.ds(j * 128, 128)], sem_u))
        copies.append(pltpu.async_copy(
            rtab_hbm.at[ridx_v.at[j]], rrows_v.at[pl.ds(j * 128, 128)], sem_r))
    for c in copies:
        c.wait()
    pltpu.sync_copy(urows_v, uout_hbm.at[pl.ds(base, BPW)])
    pltpu.sync_copy(rrows_v, rout_hbm.at[pl.ds(base, BPW)])


def _make_sc_gather():
    mesh = plsc.VectorSubcoreMesh(core_axis_name="c", subcore_axis_name="s")
    return functools.partial(
        pl.kernel, mesh=mesh,
        compiler_params=pltpu.CompilerParams(use_tc_tiling_on_sc=False),
        out_type=[jax.ShapeDtypeStruct((B, E), jnp.float32),
                  jax.ShapeDtypeStruct((B, E), jnp.float32)],
        scratch_types=[
            pltpu.VMEM((IDX_ROWS, 128), jnp.int32),
            pltpu.VMEM((IDX_ROWS, 128), jnp.int32),
            pltpu.VMEM((BPW, E), jnp.float32),
            pltpu.VMEM((BPW, E), jnp.float32),
            pltpu.SemaphoreType.DMA,
            pltpu.SemaphoreType.DMA,
        ])(_sc_gather_body)


_sc_gather_cached = None


def _sc_gather(*args):
    global _sc_gather_cached
    if _sc_gather_cached is None:
        _sc_gather_cached = _make_sc_gather()
    return _sc_gather_cached(*args)

BLK = 1024


def _mlp_body(u_ref, r_ref, t_ref, v_ref, w1_ref, b1_ref, w2_ref, b2_ref,
              w3_ref, b3_ref, out_ref):
    x = (u_ref[...] @ w1_ref[0:E, :]
         + r_ref[...] @ w1_ref[E:2 * E, :]
         + t_ref[...] @ w1_ref[2 * E:2 * E + T, :]
         + v_ref[...] @ w1_ref[2 * E + T:F, :])
    h = jnp.maximum(x + b1_ref[...], 0.0)
    h = jnp.maximum(h @ w2_ref[...] + b2_ref[...], 0.0)
    logit = jnp.sum(h * w3_ref[...], axis=1) + b3_ref[0, 0]
    out_ref[...] = (1.0 / (1.0 + jnp.exp(-logit))).reshape(1, 1, BLK)


def _mlp(u, r, t, v, W1, b1, W2, b2, W3, b3):
    n = B // BLK
    out = pl.pallas_call(
        _mlp_body,
        grid=(n,),
        in_specs=[
            pl.BlockSpec((BLK, E), lambda i: (i, 0)),
            pl.BlockSpec((BLK, E), lambda i: (i, 0)),
            pl.BlockSpec((BLK, T), lambda i: (i, 0)),
            pl.BlockSpec((BLK, V), lambda i: (i, 0)),
            pl.BlockSpec((F, H), lambda i: (0, 0)),
            pl.BlockSpec((1, H), lambda i: (0, 0)),
            pl.BlockSpec((H, H), lambda i: (0, 0)),
            pl.BlockSpec((1, H), lambda i: (0, 0)),
            pl.BlockSpec((1, H), lambda i: (0, 0)),
            pl.BlockSpec(memory_space=pltpu.SMEM),
        ],
        out_specs=pl.BlockSpec((1, 1, BLK), lambda i: (i, 0, 0)),
        out_shape=jax.ShapeDtypeStruct((n, 1, BLK), jnp.float32),
    )(u, r, t, v, W1, b1.reshape(1, H), W2, b2.reshape(1, H),
      W3.reshape(1, H), b3.reshape(1, 1))
    return out.reshape(B)


def kernel(user_indices, reel_indices, text_vectors, vision_vectors,
           user_table, reel_table, W1, b1, W2, b2, W3, b3):
    uidx = user_indices.astype(jnp.int32).reshape(B // 128, 128)
    ridx = reel_indices.astype(jnp.int32).reshape(B // 128, 128)
    u, r = _sc_gather(uidx, ridx, user_table, reel_table)
    return _mlp(u, r, text_vectors, vision_vectors, W1, b1, W2, b2, W3, b3)
